# baseline (speedup 1.0000x reference)
import functools

import jax
import jax.numpy as jnp
from jax import lax
from jax.experimental import pallas as pl
from jax.experimental.pallas import tpu as pltpu
from jax.experimental.pallas import tpu_sc as plsc

N_NODES = 10000
N_EDGES = 320000
CHUNK = 128
N_CHUNKS = 2560
E_PAD = N_CHUNKS * CHUNK
N_WORKERS = 32
CW = N_CHUNKS // N_WORKERS
NP = 10112
ROWS_PER_TILE = NP // 16
DUMMY_ROW = N_NODES
BLK = 2000


@functools.cache
def _sc_prop(W: int):
    mesh = plsc.VectorSubcoreMesh(core_axis_name="c", subcore_axis_name="s")

    NBUF = 4

    @functools.partial(
        pl.kernel,
        out_type=jax.ShapeDtypeStruct((2, NP, W), jnp.float32),
        mesh=mesh,
        scratch_types=[
            pltpu.VMEM((CW, CHUNK), jnp.int32),
            pltpu.VMEM((CW, CHUNK), jnp.int32),
            [pltpu.VMEM((CHUNK, W), jnp.float32) for _ in range(NBUF)],
            [pltpu.SemaphoreType.DMA for _ in range(NBUF)],
            [pltpu.SemaphoreType.DMA for _ in range(NBUF)],
            pltpu.VMEM_SHARED((NP, W), jnp.float32),
        ],
        compiler_params=pltpu.CompilerParams(use_tc_tiling_on_sc=False),
    )
    def prop(x_hbm, src_hbm, dst_hbm, z_hbm, out_hbm, src_v, dst_v, rows_v,
             gsems, ssems, agg_s):
        c = lax.axis_index("c")
        s = lax.axis_index("s")
        wid = c * 16 + s
        pltpu.sync_copy(src_hbm.at[pl.ds(wid * CW, CW)], src_v)
        pltpu.sync_copy(dst_hbm.at[pl.ds(wid * CW, CW)], dst_v)
        pltpu.sync_copy(z_hbm, agg_s.at[pl.ds(s * ROWS_PER_TILE, ROWS_PER_TILE)])
        for b in range(NBUF):
            pltpu.async_copy(x_hbm.at[src_v.at[b]], rows_v[b], gsems[b])
        plsc.subcore_barrier()

        def body(i, carry):
            for b in range(NBUF):
                j = i * NBUF + b
                pltpu.make_async_copy(x_hbm.at[src_v.at[j]], rows_v[b],
                                      gsems[b]).wait()
                pltpu.async_copy(rows_v[b], agg_s.at[dst_v.at[j]], ssems[b],
                                 add=True)

                @pl.when(j + NBUF < CW)
                def _():
                    pltpu.make_async_copy(rows_v[b], agg_s.at[dst_v.at[j]],
                                          ssems[b]).wait()
                    pltpu.async_copy(x_hbm.at[src_v.at[j + NBUF]], rows_v[b],
                                     gsems[b])
            return carry

        lax.fori_loop(0, CW // NBUF, body, 0)
        for b in range(NBUF):
            pltpu.make_async_copy(rows_v[b], agg_s.at[dst_v.at[CW - NBUF + b]],
                                  ssems[b]).wait()
        plsc.subcore_barrier()
        pltpu.sync_copy(agg_s.at[pl.ds(s * ROWS_PER_TILE, ROWS_PER_TILE)],
                        out_hbm.at[c, pl.ds(s * ROWS_PER_TILE, ROWS_PER_TILE)])

    return prop


def _norm_blk(degp_blk):
    deg = degp_blk[0, :, 0:1] + degp_blk[1, :, 0:1]
    return lax.rsqrt(jnp.maximum(deg, 1.0))


def _tc_matmul_scale(h, Wc, degp, W):
    din = h.shape[1]

    def body(h_ref, w_ref, d_ref, t_ref, s2_ref):
        T = jnp.dot(h_ref[...], w_ref[...], preferred_element_type=jnp.float32)
        t_ref[...] = T
        s2_ref[...] = T[:, 2 * W:3 * W] * _norm_blk(d_ref[...])

    return pl.pallas_call(
        body,
        grid=(N_NODES // BLK,),
        in_specs=[
            pl.BlockSpec((BLK, din), lambda i: (i, 0)),
            pl.BlockSpec((din, 3 * W), lambda i: (0, 0)),
            pl.BlockSpec((2, BLK, 16), lambda i: (0, i, 0)),
        ],
        out_specs=[
            pl.BlockSpec((BLK, 3 * W), lambda i: (i, 0)),
            pl.BlockSpec((BLK, W), lambda i: (i, 0)),
        ],
        out_shape=[
            jax.ShapeDtypeStruct((N_NODES, 3 * W), jnp.float32),
            jax.ShapeDtypeStruct((N_NODES, W), jnp.float32),
        ],
    )(h, Wc, degp)


def _tc_mid(T, degp, Ap, W):
    def body(t_ref, d_ref, a_ref, o_ref):
        norm = _norm_blk(d_ref[...])
        a = a_ref[0] + a_ref[1]
        o_ref[...] = norm * (t_ref[:, W:2 * W] + norm * a)

    return pl.pallas_call(
        body,
        grid=(N_NODES // BLK,),
        in_specs=[
            pl.BlockSpec((BLK, 3 * W), lambda i: (i, 0)),
            pl.BlockSpec((2, BLK, 16), lambda i: (0, i, 0)),
            pl.BlockSpec((2, BLK, W), lambda i: (0, i, 0)),
        ],
        out_specs=pl.BlockSpec((BLK, W), lambda i: (i, 0)),
        out_shape=jax.ShapeDtypeStruct((N_NODES, W), jnp.float32),
    )(T, degp, Ap)


def _tc_out(T, degp, Ap, b, W, relu):
    def body(t_ref, d_ref, a_ref, b_ref, o_ref):
        norm = _norm_blk(d_ref[...])
        a = a_ref[0] + a_ref[1]
        o = t_ref[:, 0:W] + norm * a + b_ref[...]
        o_ref[...] = jnp.maximum(o, 0.0) if relu else o

    return pl.pallas_call(
        body,
        grid=(N_NODES // BLK,),
        in_specs=[
            pl.BlockSpec((BLK, 3 * W), lambda i: (i, 0)),
            pl.BlockSpec((2, BLK, 16), lambda i: (0, i, 0)),
            pl.BlockSpec((2, BLK, W), lambda i: (0, i, 0)),
            pl.BlockSpec((1, W), lambda i: (0, 0)),
        ],
        out_specs=pl.BlockSpec((BLK, W), lambda i: (i, 0)),
        out_shape=jax.ShapeDtypeStruct((N_NODES, W), jnp.float32),
    )(T, degp, Ap, b.reshape(1, W))


def _layer(h, Wmat, b, degp, src2d, dst2d, din, dout, relu):
    Wc = jnp.concatenate(
        [Wmat[0:din], Wmat[din:2 * din], Wmat[2 * din:3 * din]], axis=1)
    z = jnp.zeros((ROWS_PER_TILE, dout), jnp.float32)
    T, S2 = _tc_matmul_scale(h, Wc, degp, dout)
    A2p = _sc_prop(dout)(S2, src2d, dst2d, z)
    Vs = _tc_mid(T, degp, A2p, dout)
    A1p = _sc_prop(dout)(Vs, src2d, dst2d, z)
    return _tc_out(T, degp, A1p, b, dout, relu)


def kernel(x, edge_index, W1, b1, W2, b2, W3, b3):
    src = edge_index[0].astype(jnp.int32)
    dst = edge_index[1].astype(jnp.int32)
    pad = E_PAD - N_EDGES
    pad_src = jnp.arange(pad, dtype=jnp.int32) % N_NODES
    pad_dst = DUMMY_ROW + (jnp.arange(pad, dtype=jnp.int32) % (NP - N_NODES))
    src2d = jnp.concatenate([src, pad_src]).reshape(N_CHUNKS, CHUNK)
    dst2d = jnp.concatenate([dst, pad_dst]).reshape(N_CHUNKS, CHUNK)

    ones16 = jnp.ones((N_NODES, 16), jnp.float32)
    z16 = jnp.zeros((ROWS_PER_TILE, 16), jnp.float32)
    degp = _sc_prop(16)(ones16, src2d, dst2d, z16)

    h = _layer(x, W1, b1, degp, src2d, dst2d, 128, 32, relu=True)
    h = _layer(h, W2, b2, degp, src2d, dst2d, 32, 32, relu=True)
    h = _layer(h, W3, b3, degp, src2d, dst2d, 32, 16, relu=False)
    return h


# speedup vs baseline: 1.0759x; 1.0759x over previous
import functools

import jax
import jax.numpy as jnp
import numpy as np
from jax import lax
from jax.experimental import pallas as pl
from jax.experimental.pallas import tpu as pltpu
from jax.experimental.pallas import tpu_sc as plsc

N_NODES = 10000
N_EDGES = 320000
CHUNK = 128
N_CHUNKS = 2560
E_PAD = N_CHUNKS * CHUNK
N_WORKERS = 32
CW = N_CHUNKS // N_WORKERS
NP = 10112
ROWS_PER_TILE = NP // 16
DUMMY_ROW = N_NODES
BLK = 2000
NBUF = 4

_PAD = E_PAD - N_EDGES
_PAD_SRC = np.arange(_PAD, dtype=np.int32) % N_NODES
_PAD_DST = (DUMMY_ROW + np.arange(_PAD, dtype=np.int32) % (NP - N_NODES)
            ).astype(np.int32)


def _mesh():
    return plsc.VectorSubcoreMesh(core_axis_name="c", subcore_axis_name="s")


@functools.cache
def _sc_prop(W: int):
    @functools.partial(
        pl.kernel,
        out_type=jax.ShapeDtypeStruct((2, NP, W), jnp.float32),
        mesh=_mesh(),
        scratch_types=[
            pltpu.VMEM((CW, CHUNK), jnp.int32),
            pltpu.VMEM((CW, CHUNK), jnp.int32),
            [pltpu.VMEM((CHUNK, W), jnp.float32) for _ in range(NBUF)],
            [pltpu.SemaphoreType.DMA for _ in range(NBUF)],
            [pltpu.SemaphoreType.DMA for _ in range(NBUF)],
            pltpu.VMEM_SHARED((NP, W), jnp.float32),
        ],
        compiler_params=pltpu.CompilerParams(use_tc_tiling_on_sc=False),
    )
    def prop(x_hbm, src_hbm, dst_hbm, z_hbm, out_hbm, src_v, dst_v, rows_v,
             gsems, ssems, agg_s):
        c = lax.axis_index("c")
        s = lax.axis_index("s")
        wid = c * 16 + s
        pltpu.sync_copy(src_hbm.at[pl.ds(wid * CW, CW)], src_v)
        pltpu.sync_copy(dst_hbm.at[pl.ds(wid * CW, CW)], dst_v)
        pltpu.sync_copy(z_hbm, agg_s.at[pl.ds(s * ROWS_PER_TILE, ROWS_PER_TILE)])
        for b in range(NBUF):
            pltpu.async_copy(x_hbm.at[src_v.at[b]], rows_v[b], gsems[b])
        plsc.subcore_barrier()

        def body(i, carry):
            for b in range(NBUF):
                j = i * NBUF + b
                pltpu.make_async_copy(x_hbm.at[src_v.at[j]], rows_v[b],
                                      gsems[b]).wait()
                pltpu.async_copy(rows_v[b], agg_s.at[dst_v.at[j]], ssems[b],
                                 add=True)

                @pl.when(j + NBUF < CW)
                def _():
                    pltpu.make_async_copy(rows_v[b], agg_s.at[dst_v.at[j]],
                                          ssems[b]).wait()
                    pltpu.async_copy(x_hbm.at[src_v.at[j + NBUF]], rows_v[b],
                                     gsems[b])
            return carry

        lax.fori_loop(0, CW // NBUF, body, 0)
        for b in range(NBUF):
            pltpu.make_async_copy(rows_v[b], agg_s.at[dst_v.at[CW - NBUF + b]],
                                  ssems[b]).wait()
        plsc.subcore_barrier()
        pltpu.sync_copy(agg_s.at[pl.ds(s * ROWS_PER_TILE, ROWS_PER_TILE)],
                        out_hbm.at[c, pl.ds(s * ROWS_PER_TILE, ROWS_PER_TILE)])

    return prop


@functools.cache
def _sc_deg():
    W = 16

    @functools.partial(
        pl.kernel,
        out_type=jax.ShapeDtypeStruct((2, NP, W), jnp.float32),
        mesh=_mesh(),
        scratch_types=[
            pltpu.VMEM((CW, CHUNK), jnp.int32),
            pltpu.VMEM((CHUNK, W), jnp.float32),
            [pltpu.SemaphoreType.DMA for _ in range(NBUF)],
            pltpu.VMEM_SHARED((NP, W), jnp.float32),
        ],
        compiler_params=pltpu.CompilerParams(use_tc_tiling_on_sc=False),
    )
    def deg(ones_hbm, dst_hbm, z_hbm, out_hbm, dst_v, ones_v, ssems, agg_s):
        c = lax.axis_index("c")
        s = lax.axis_index("s")
        wid = c * 16 + s
        pltpu.sync_copy(dst_hbm.at[pl.ds(wid * CW, CW)], dst_v)
        pltpu.sync_copy(ones_hbm, ones_v)
        pltpu.sync_copy(z_hbm, agg_s.at[pl.ds(s * ROWS_PER_TILE, ROWS_PER_TILE)])
        plsc.subcore_barrier()

        def body(i, carry):
            for b in range(NBUF):
                j = i * NBUF + b

                @pl.when(j >= NBUF)
                def _():
                    pltpu.make_async_copy(ones_v, agg_s.at[dst_v.at[j - NBUF]],
                                          ssems[b]).wait()

                pltpu.async_copy(ones_v, agg_s.at[dst_v.at[j]], ssems[b],
                                 add=True)
            return carry

        lax.fori_loop(0, CW // NBUF, body, 0)
        for b in range(NBUF):
            pltpu.make_async_copy(ones_v, agg_s.at[dst_v.at[CW - NBUF + b]],
                                  ssems[b]).wait()
        plsc.subcore_barrier()
        pltpu.sync_copy(agg_s.at[pl.ds(s * ROWS_PER_TILE, ROWS_PER_TILE)],
                        out_hbm.at[c, pl.ds(s * ROWS_PER_TILE, ROWS_PER_TILE)])

    return deg


def _tc_matmul(h, Wc, W):
    din, dcols = Wc.shape

    def body(h_ref, w_ref, t0_ref, t1_ref, t2_ref):
        T = jnp.dot(h_ref[...], w_ref[...], preferred_element_type=jnp.float32)
        t0_ref[...] = T[:, 0:W]
        t1_ref[...] = T[:, W:2 * W]
        t2_ref[...] = T[:, 2 * W:3 * W]

    return pl.pallas_call(
        body,
        grid=(N_NODES // BLK,),
        in_specs=[
            pl.BlockSpec((BLK, din), lambda i: (i, 0)),
            pl.BlockSpec((din, dcols), lambda i: (0, 0)),
        ],
        out_specs=[pl.BlockSpec((BLK, W), lambda i: (i, 0))] * 3,
        out_shape=[jax.ShapeDtypeStruct((N_NODES, W), jnp.float32)] * 3,
    )(h, Wc)


def _tc_norm_scale(degp, t2, W):
    def body(d_ref, t2_ref, n_ref, s2_ref):
        deg = d_ref[0, :, 0:1] + d_ref[1, :, 0:1]
        n = lax.rsqrt(jnp.maximum(deg, 1.0))
        n_ref[...] = n
        s2_ref[...] = t2_ref[...] * n

    return pl.pallas_call(
        body,
        grid=(N_NODES // BLK,),
        in_specs=[
            pl.BlockSpec((2, BLK, 16), lambda i: (0, i, 0)),
            pl.BlockSpec((BLK, W), lambda i: (i, 0)),
        ],
        out_specs=[
            pl.BlockSpec((BLK, 1), lambda i: (i, 0)),
            pl.BlockSpec((BLK, W), lambda i: (i, 0)),
        ],
        out_shape=[
            jax.ShapeDtypeStruct((N_NODES, 1), jnp.float32),
            jax.ShapeDtypeStruct((N_NODES, W), jnp.float32),
        ],
    )(degp, t2)


def _tc_mid(t1, norm, Ap, W):
    def body(t1_ref, n_ref, a_ref, o_ref):
        n = n_ref[...]
        o_ref[...] = n * (t1_ref[...] + n * (a_ref[0] + a_ref[1]))

    return pl.pallas_call(
        body,
        grid=(N_NODES // BLK,),
        in_specs=[
            pl.BlockSpec((BLK, W), lambda i: (i, 0)),
            pl.BlockSpec((BLK, 1), lambda i: (i, 0)),
            pl.BlockSpec((2, BLK, W), lambda i: (0, i, 0)),
        ],
        out_specs=pl.BlockSpec((BLK, W), lambda i: (i, 0)),
        out_shape=jax.ShapeDtypeStruct((N_NODES, W), jnp.float32),
    )(t1, norm, Ap)


def _tc_out_matmul(t0, norm, Ap, b, Wc_next, W, W_next):
    din, dcols = Wc_next.shape

    def body(t0_ref, n_ref, a_ref, b_ref, w_ref,
             o0_ref, o1_ref, o2_ref, s2_ref):
        n = n_ref[...]
        h = t0_ref[...] + n * (a_ref[0] + a_ref[1]) + b_ref[...]
        h = jnp.maximum(h, 0.0)
        Tn = jnp.dot(h, w_ref[...], preferred_element_type=jnp.float32)
        o0_ref[...] = Tn[:, 0:W_next]
        o1_ref[...] = Tn[:, W_next:2 * W_next]
        t2 = Tn[:, 2 * W_next:3 * W_next]
        o2_ref[...] = t2
        s2_ref[...] = t2 * n

    return pl.pallas_call(
        body,
        grid=(N_NODES // BLK,),
        in_specs=[
            pl.BlockSpec((BLK, W), lambda i: (i, 0)),
            pl.BlockSpec((BLK, 1), lambda i: (i, 0)),
            pl.BlockSpec((2, BLK, W), lambda i: (0, i, 0)),
            pl.BlockSpec((1, W), lambda i: (0, 0)),
            pl.BlockSpec((din, dcols), lambda i: (0, 0)),
        ],
        out_specs=[pl.BlockSpec((BLK, W_next), lambda i: (i, 0))] * 4,
        out_shape=[jax.ShapeDtypeStruct((N_NODES, W_next), jnp.float32)] * 4,
    )(t0, norm, Ap, b.reshape(1, W), Wc_next)


def _tc_out(t0, norm, Ap, b, W):
    def body(t0_ref, n_ref, a_ref, b_ref, o_ref):
        n = n_ref[...]
        o_ref[...] = t0_ref[...] + n * (a_ref[0] + a_ref[1]) + b_ref[...]

    return pl.pallas_call(
        body,
        grid=(N_NODES // BLK,),
        in_specs=[
            pl.BlockSpec((BLK, W), lambda i: (i, 0)),
            pl.BlockSpec((BLK, 1), lambda i: (i, 0)),
            pl.BlockSpec((2, BLK, W), lambda i: (0, i, 0)),
            pl.BlockSpec((1, W), lambda i: (0, 0)),
        ],
        out_specs=pl.BlockSpec((BLK, W), lambda i: (i, 0)),
        out_shape=jax.ShapeDtypeStruct((N_NODES, W), jnp.float32),
    )(t0, norm, Ap, b.reshape(1, W))


def _wcat(Wmat, din):
    return jnp.concatenate(
        [Wmat[0:din], Wmat[din:2 * din], Wmat[2 * din:3 * din]], axis=1)


def kernel(x, edge_index, W1, b1, W2, b2, W3, b3):
    src = edge_index[0].astype(jnp.int32)
    dst = edge_index[1].astype(jnp.int32)
    src2d = jnp.concatenate([src, jnp.asarray(_PAD_SRC)]).reshape(
        N_CHUNKS, CHUNK)
    dst2d = jnp.concatenate([dst, jnp.asarray(_PAD_DST)]).reshape(
        N_CHUNKS, CHUNK)

    ones = jnp.ones((CHUNK, 16), jnp.float32)
    z16 = jnp.zeros((ROWS_PER_TILE, 16), jnp.float32)
    z32 = jnp.zeros((ROWS_PER_TILE, 32), jnp.float32)

    t0, t1, t2 = _tc_matmul(x, _wcat(W1, 128), 32)
    degp = _sc_deg()(ones, dst2d, z16)
    norm, S2 = _tc_norm_scale(degp, t2, 32)

    A2p = _sc_prop(32)(S2, src2d, dst2d, z32)
    Vs = _tc_mid(t1, norm, A2p, 32)
    A1p = _sc_prop(32)(Vs, src2d, dst2d, z32)
    t0, t1, t2, S2 = _tc_out_matmul(t0, norm, A1p, b1, _wcat(W2, 32), 32, 32)
    A2p = _sc_prop(32)(S2, src2d, dst2d, z32)
    Vs = _tc_mid(t1, norm, A2p, 32)
    A1p = _sc_prop(32)(Vs, src2d, dst2d, z32)
    t0, t1, t2, S2 = _tc_out_matmul(t0, norm, A1p, b2, _wcat(W3, 32), 32, 16)
    A2p = _sc_prop(16)(S2, src2d, dst2d, z16)
    Vs = _tc_mid(t1, norm, A2p, 16)
    A1p = _sc_prop(16)(Vs, src2d, dst2d, z16)
    return _tc_out(t0, norm, A1p, b3, 16)
